# baseline (speedup 1.0000x reference)
import functools

import jax
import jax.numpy as jnp
from jax import lax
from jax.experimental import pallas as pl
from jax.experimental.pallas import tpu as pltpu
from jax.experimental.pallas import tpu_sc as plsc

BATCH = 16384
HIST = 50
H_HALF = HIST // 2
D = 64
B_TOTAL = BATCH * H_HALF
NC = 2
NS = 16
NW = NC * NS
B_PER_W = B_TOTAL // NW
CHUNK = 256
N_CHUNKS = B_PER_W // CHUNK
K = 1
NG = N_CHUNKS // K
GROUP_ROWS = K * CHUNK

_mesh = plsc.VectorSubcoreMesh(core_axis_name="c", subcore_axis_name="s")


@functools.partial(
    pl.kernel,
    mesh=_mesh,
    out_type=jax.ShapeDtypeStruct((B_TOTAL, D), jnp.float32),
    scratch_types=[
        pltpu.VMEM((N_CHUNKS, CHUNK), jnp.int32),
        pltpu.VMEM((GROUP_ROWS, D), jnp.float32),
        pltpu.VMEM((GROUP_ROWS, D), jnp.float32),
        pltpu.SemaphoreType.DMA,
        pltpu.SemaphoreType.DMA,
        pltpu.SemaphoreType.DMA,
        pltpu.SemaphoreType.DMA,
    ],
    compiler_params=pltpu.CompilerParams(use_tc_tiling_on_sc=False),
)
def _sc_gather(idx_hbm, table_hbm, out_hbm, idx_v, rows0, rows1,
               g0, g1, w0, w1):
    wid = lax.axis_index("s") * NC + lax.axis_index("c")
    base = wid * B_PER_W
    rows = [rows0, rows1]
    gsem = [g0, g1]
    wsem = [w0, w1]

    pltpu.sync_copy(idx_hbm.at[pl.ds(wid * N_CHUNKS, N_CHUNKS)], idx_v)

    def fire(group, buf, sem):
        for b in range(K):
            pltpu.async_copy(
                table_hbm.at[idx_v.at[group * K + b]],
                buf.at[pl.ds(b * CHUNK, CHUNK)],
                sem,
            )

    def drain(group, buf, sem):
        for b in range(K):
            pltpu.make_async_copy(
                table_hbm.at[idx_v.at[group * K + b]],
                buf.at[pl.ds(b * CHUNK, CHUNK)],
                sem,
            ).wait()

    def writeback_copy(group, buf, sem):
        return pltpu.make_async_copy(
            buf, out_hbm.at[pl.ds(base + group * GROUP_ROWS, GROUP_ROWS)], sem)

    def start_writeback(group, buf, sem):
        pltpu.async_copy(
            buf, out_hbm.at[pl.ds(base + group * GROUP_ROWS, GROUP_ROWS)], sem)

    NP = NG // 2

    fire(0, rows[0], gsem[0])

    def body(p, carry):
        g = 2 * p

        @pl.when(p > 0)
        def _wait_wb1():
            writeback_copy(g - 1, rows[1], wsem[1]).wait()

        fire(g + 1, rows[1], gsem[1])
        drain(g, rows[0], gsem[0])
        start_writeback(g, rows[0], wsem[0])

        @pl.when(p + 1 < NP)
        def _fire_next_pair():
            writeback_copy(g, rows[0], wsem[0]).wait()
            fire(g + 2, rows[0], gsem[0])

        drain(g + 1, rows[1], gsem[1])
        start_writeback(g + 1, rows[1], wsem[1])
        return carry

    lax.fori_loop(0, NP, body, 0)

    writeback_copy(NG - 2, rows[0], wsem[0]).wait()
    writeback_copy(NG - 1, rows[1], wsem[1]).wait()


def kernel(indices, table):
    idx32 = indices.astype(jnp.int32)
    idx_a = idx32[:, :H_HALF].reshape(B_TOTAL // CHUNK, CHUNK)
    idx_b = idx32[:, H_HALF:].reshape(B_TOTAL // CHUNK, CHUNK)
    out_a = _sc_gather(idx_a, table).reshape(BATCH, H_HALF, D)
    out_b = _sc_gather(idx_b, table).reshape(BATCH, H_HALF, D)
    return jnp.concatenate([out_a, out_b], axis=1)


# speedup vs baseline: 1.0904x; 1.0904x over previous
import functools

import jax
import jax.numpy as jnp
from jax import lax
from jax.experimental import pallas as pl
from jax.experimental.pallas import tpu as pltpu
from jax.experimental.pallas import tpu_sc as plsc

BATCH = 16384
HIST = 50
D = 64
B_TOTAL = BATCH * HIST
NC = 2
NS = 16
NW = NC * NS
B_PER_W = B_TOTAL // NW
CHUNK = 128
N_CHUNKS = B_PER_W // CHUNK
K = 4
NG = N_CHUNKS // K
GROUP_ROWS = K * CHUNK

_mesh = plsc.VectorSubcoreMesh(core_axis_name="c", subcore_axis_name="s")


@functools.partial(
    pl.kernel,
    mesh=_mesh,
    out_type=jax.ShapeDtypeStruct((B_TOTAL, D), jnp.float32),
    scratch_types=[
        pltpu.VMEM((N_CHUNKS, CHUNK), jnp.int32),
        pltpu.VMEM((GROUP_ROWS, D), jnp.float32),
        pltpu.VMEM((GROUP_ROWS, D), jnp.float32),
        pltpu.SemaphoreType.DMA,
        pltpu.SemaphoreType.DMA,
        pltpu.SemaphoreType.DMA,
        pltpu.SemaphoreType.DMA,
    ],
    compiler_params=pltpu.CompilerParams(use_tc_tiling_on_sc=False),
)
def _sc_gather(idx_hbm, table_hbm, out_hbm, idx_v, rows0, rows1,
               g0, g1, w0, w1):
    wid = lax.axis_index("s") * NC + lax.axis_index("c")
    base = wid * B_PER_W
    rows = [rows0, rows1]
    gsem = [g0, g1]
    wsem = [w0, w1]

    pltpu.sync_copy(idx_hbm.at[pl.ds(wid * N_CHUNKS, N_CHUNKS)], idx_v)

    def fire(group, buf, sem):
        for b in range(K):
            pltpu.async_copy(
                table_hbm.at[idx_v.at[group * K + b]],
                buf.at[pl.ds(b * CHUNK, CHUNK)],
                sem,
            )

    def drain(group, buf, sem):
        for b in range(K):
            pltpu.make_async_copy(
                table_hbm.at[idx_v.at[group * K + b]],
                buf.at[pl.ds(b * CHUNK, CHUNK)],
                sem,
            ).wait()

    def writeback_copy(group, buf, sem):
        return pltpu.make_async_copy(
            buf, out_hbm.at[pl.ds(base + group * GROUP_ROWS, GROUP_ROWS)], sem)

    def start_writeback(group, buf, sem):
        pltpu.async_copy(
            buf, out_hbm.at[pl.ds(base + group * GROUP_ROWS, GROUP_ROWS)], sem)

    NP = NG // 2

    fire(0, rows[0], gsem[0])

    def body(p, carry):
        g = 2 * p

        @pl.when(p > 0)
        def _wait_wb1():
            writeback_copy(g - 1, rows[1], wsem[1]).wait()

        fire(g + 1, rows[1], gsem[1])
        drain(g, rows[0], gsem[0])
        start_writeback(g, rows[0], wsem[0])

        @pl.when(p + 1 < NP)
        def _fire_next_pair():
            writeback_copy(g, rows[0], wsem[0]).wait()
            fire(g + 2, rows[0], gsem[0])

        drain(g + 1, rows[1], gsem[1])
        start_writeback(g + 1, rows[1], wsem[1])
        return carry

    lax.fori_loop(0, NP, body, 0)

    writeback_copy(NG - 2, rows[0], wsem[0]).wait()
    writeback_copy(NG - 1, rows[1], wsem[1]).wait()


def kernel(indices, table):
    idx = indices.reshape(B_TOTAL // CHUNK, CHUNK).astype(jnp.int32)
    out = _sc_gather(idx, table)
    return out.reshape(BATCH, HIST, D)
